# Initial kernel scaffold; baseline (speedup 1.0000x reference)
import functools

import jax
import jax.numpy as jnp
from jax import lax
from jax.experimental import pallas as pl
from jax.experimental.pallas import tpu as pltpu
from jax.experimental.pallas import tpu_sc as plsc

VOCAB = 100000
H = 64
S = 200
B = 1024
N = B * S
EPS = 1e-12

NC = 2
NS = 16
NW = NC * NS
PER_W = N // NW
G = 64
NBLK = PER_W // G
L = 16


def _rsqrt(x):
    xi = plsc.bitcast(x, jnp.int32)
    yi = jnp.int32(0x5F3759DF) - (xi >> 1)
    y = plsc.bitcast(yi, jnp.float32)
    xh = x * 0.5
    for _ in range(3):
        y = y * (1.5 - xh * y * y)
    return y


def _body(tok_hbm, sem_hbm, spat_hbm, gamma_hbm, beta_hbm, out_hbm,
          idx_v, spat_v, gamma_v, beta_v, rows_v, out_v, gsem):
    wid = lax.axis_index("s") * NC + lax.axis_index("c")
    base_blk = wid * NBLK
    row_base = wid * PER_W

    pltpu.sync_copy(tok_hbm.at[pl.ds(base_blk, NBLK)], idx_v)
    pltpu.sync_copy(spat_hbm, spat_v)
    pltpu.sync_copy(gamma_hbm, gamma_v)
    pltpu.sync_copy(beta_hbm, beta_v)

    iota = lax.iota(jnp.int32, L)

    def step(k, carry):
        del carry
        pltpu.async_copy(sem_hbm.at[idx_v.at[k]], rows_v, gsem).wait()
        g0 = k * G
        for t in range(G // L):
            r_idx = iota + (t * L)
            pos = jnp.remainder(iota + (g0 + t * L), S)
            s = jnp.zeros((L,), jnp.float32)
            s2 = jnp.zeros((L,), jnp.float32)
            for h in range(H):
                hs = jnp.full((L,), h, jnp.int32)
                c = (plsc.load_gather(rows_v, [r_idx, hs])
                     + plsc.load_gather(spat_v, [pos, hs]))
                plsc.store_scatter(out_v, [r_idx, hs], c)
                s = s + c
                s2 = s2 + c * c
            mean = s * (1.0 / H)
            var = s2 * (1.0 / H) - mean * mean
            rstd = _rsqrt(var + EPS)
            for h in range(H):
                hs = jnp.full((L,), h, jnp.int32)
                c = plsc.load_gather(out_v, [r_idx, hs])
                gm = plsc.load_gather(gamma_v, [hs])
                bt = plsc.load_gather(beta_v, [hs])
                o = (c - mean) * (rstd * gm) + bt
                plsc.store_scatter(out_v, [r_idx, hs], o)
        pltpu.sync_copy(out_v, out_hbm.at[pl.ds(row_base + g0, G)])
        return 0

    lax.fori_loop(0, NBLK, step, 0)


def kernel(token_idx, semantic_table, spatial_table, gamma, beta):
    tok2d = token_idx.reshape(N).astype(jnp.int32).reshape(NW * NBLK, G)
    spat = spatial_table[:S]
    mesh = plsc.VectorSubcoreMesh(core_axis_name="c", subcore_axis_name="s")
    f = pl.kernel(
        _body,
        out_type=jax.ShapeDtypeStruct((N, H), jnp.float32),
        mesh=mesh,
        scratch_types=[
            pltpu.VMEM((NBLK, G), jnp.int32),
            pltpu.VMEM((S, H), jnp.float32),
            pltpu.VMEM((H,), jnp.float32),
            pltpu.VMEM((H,), jnp.float32),
            pltpu.VMEM((G, H), jnp.float32),
            pltpu.VMEM((G, H), jnp.float32),
            pltpu.SemaphoreType.DMA,
        ],
    )
    out = f(tok2d, semantic_table, spat, gamma, beta)
    return out.reshape(B, S, H)


# speedup vs baseline: 1.6147x; 1.6147x over previous
import jax
import jax.numpy as jnp
from jax import lax
from jax.experimental import pallas as pl
from jax.experimental.pallas import tpu as pltpu
from jax.experimental.pallas import tpu_sc as plsc

H = 64
S = 200
B = 1024
N = B * S
EPS = 1e-12

NC = 2
NS = 16
NW = NC * NS
PER_W = N // NW
G = 64
NBLK = PER_W // G
L = 16
Q = H // L


def _rsqrt(x):
    xi = plsc.bitcast(x, jnp.int32)
    yi = jnp.int32(0x5F3759DF) - (xi >> 1)
    y = plsc.bitcast(yi, jnp.float32)
    xh = x * 0.5
    for _ in range(2):
        y = y * (1.5 - xh * y * y)
    return y


def _body(tok_hbm, sem_hbm, spat_hbm, gamma_hbm, beta_hbm, out_hbm,
          idx_v, spat_v, gamma_v, beta_v, rows_v, out_v, gsem):
    wid = lax.axis_index("s") * NC + lax.axis_index("c")
    row_base = wid * PER_W

    pltpu.sync_copy(tok_hbm.at[pl.ds(row_base, PER_W)], idx_v)
    pltpu.sync_copy(spat_hbm, spat_v)
    pltpu.sync_copy(gamma_hbm, gamma_v)
    pltpu.sync_copy(beta_hbm, beta_v)

    def blk(k, carry):
        g0 = k * G
        pltpu.async_copy(sem_hbm.at[idx_v.at[pl.ds(g0, G)]], rows_v, gsem).wait()

        def chunk(cc, c2):
            c0 = cc * L
            gq = [gamma_v[pl.ds(q * L, L)] for q in range(Q)]
            bq = [beta_v[pl.ds(q * L, L)] for q in range(Q)]
            for r in range(L):
                rr = c0 + r
                sb = jnp.remainder(g0 + rr, S) * H
                c = [rows_v[rr, pl.ds(q * L, L)] + spat_v[pl.ds(sb + q * L, L)]
                     for q in range(Q)]
                sv = (c[0] + c[1]) + (c[2] + c[3])
                s2 = ((c[0] * c[0] + c[1] * c[1])
                      + (c[2] * c[2] + c[3] * c[3]))
                tot = jnp.full((L,), lax.reduce_sum_p.bind(sv, axes=(0,)),
                               jnp.float32)
                tot2 = jnp.full((L,), lax.reduce_sum_p.bind(s2, axes=(0,)),
                                jnp.float32)
                mean = tot * (1.0 / H)
                var = tot2 * (1.0 / H) - mean * mean
                rstd = _rsqrt(var + EPS)
                ob = rr * H
                for q in range(Q):
                    o = (c[q] - mean) * (rstd * gq[q]) + bq[q]
                    out_v[pl.ds(ob + q * L, L)] = o
            return c2

        lax.fori_loop(0, G // L, chunk, 0)
        pltpu.sync_copy(out_v, out_hbm.at[pl.ds((row_base + g0) * H, G * H)])
        return carry

    lax.fori_loop(0, NBLK, blk, 0)


def kernel(token_idx, semantic_table, spatial_table, gamma, beta):
    tok1d = token_idx.reshape(N).astype(jnp.int32)
    spat = spatial_table[:S].reshape(S * H)
    mesh = plsc.VectorSubcoreMesh(core_axis_name="c", subcore_axis_name="s")
    f = pl.kernel(
        _body,
        out_type=jax.ShapeDtypeStruct((N * H,), jnp.float32),
        mesh=mesh,
        compiler_params=pltpu.CompilerParams(
            use_tc_tiling_on_sc=False, needs_layout_passes=False),
        scratch_types=[
            pltpu.VMEM((PER_W,), jnp.int32),
            pltpu.VMEM((S * H,), jnp.float32),
            pltpu.VMEM((H,), jnp.float32),
            pltpu.VMEM((H,), jnp.float32),
            pltpu.VMEM((G, H), jnp.float32),
            pltpu.VMEM((G * H,), jnp.float32),
            pltpu.SemaphoreType.DMA,
        ],
    )
    out = f(tok1d, semantic_table, spat, gamma, beta)
    return out.reshape(B, S, H)
